# Initial kernel scaffold; baseline (speedup 1.0000x reference)
import jax
import jax.numpy as jnp
from jax.experimental import pallas as pl


def kernel(x, Wr, br, sW1, sb1, sW2, sb2, rW1, rb1, rW2, rb2):
    raise NotImplementedError("write your pallas kernel here")


# speedup vs baseline: 1.4840x; 1.4840x over previous
import functools

import jax
import jax.numpy as jnp
from jax.experimental import pallas as pl
from jax.experimental.pallas import tpu as pltpu

T = 2048
D = 768
E = 16
K = 2


def _moe_body(x_ref, Wr_ref, br_ref, sW1_ref, sb1_ref, sW2_ref, sb2_ref,
              rW1_ref, rb1_ref, rW2_ref, rb2_ref, out_ref, acc_ref, gates_ref):
    e = pl.program_id(0)
    xt = x_ref[...]

    @pl.when(e == 0)
    def _prologue():
        logits = jnp.dot(xt, Wr_ref[...], preferred_element_type=jnp.float32)
        logits = logits + br_ref[0]
        m = jnp.max(logits, axis=1, keepdims=True)
        ex = jnp.exp(logits - m)
        probs = ex / jnp.sum(ex, axis=1, keepdims=True)
        col = jax.lax.broadcasted_iota(jnp.int32, (T, E), 1)
        m1 = jnp.max(probs, axis=1, keepdims=True)
        i1 = jnp.min(jnp.where(probs == m1, col, E), axis=1, keepdims=True)
        p2 = jnp.where(col == i1, -1.0, probs)
        m2 = jnp.max(p2, axis=1, keepdims=True)
        i2 = jnp.min(jnp.where(p2 == m2, col, E), axis=1, keepdims=True)
        gates_ref[...] = jnp.where(col == i1, m1, 0.0) + jnp.where(col == i2, m2, 0.0)
        sh = jnp.maximum(
            jnp.dot(xt, sW1_ref[...], preferred_element_type=jnp.float32) + sb1_ref[0],
            0.0)
        acc_ref[...] = (
            jnp.dot(sh, sW2_ref[...], preferred_element_type=jnp.float32) + sb2_ref[0])

    h = jnp.maximum(
        jnp.dot(xt, rW1_ref[0], preferred_element_type=jnp.float32) + rb1_ref[0, 0], 0.0)
    eo = jnp.dot(h, rW2_ref[0], preferred_element_type=jnp.float32) + rb2_ref[0, 0]
    col = jax.lax.broadcasted_iota(jnp.int32, (T, E), 1)
    g = jnp.sum(jnp.where(col == e, gates_ref[...], 0.0), axis=1, keepdims=True)
    acc_ref[...] += eo * g

    @pl.when(e == E - 1)
    def _epilogue():
        out_ref[...] = acc_ref[...]


@functools.partial(jax.jit, static_argnames=())
def kernel(x, Wr, br, sW1, sb1, sW2, sb2, rW1, rb1, rW2, rb2):
    b, l, d = x.shape
    xt = x.reshape(b * l, d)
    out = pl.pallas_call(
        _moe_body,
        grid=(E,),
        in_specs=[
            pl.BlockSpec((T, D), lambda e: (0, 0)),
            pl.BlockSpec((D, E), lambda e: (0, 0)),
            pl.BlockSpec((1, E), lambda e: (0, 0)),
            pl.BlockSpec((D, D), lambda e: (0, 0)),
            pl.BlockSpec((1, D), lambda e: (0, 0)),
            pl.BlockSpec((D, D), lambda e: (0, 0)),
            pl.BlockSpec((1, D), lambda e: (0, 0)),
            pl.BlockSpec((1, D, D), lambda e: (e, 0, 0)),
            pl.BlockSpec((1, 1, D), lambda e: (e, 0, 0)),
            pl.BlockSpec((1, D, D), lambda e: (e, 0, 0)),
            pl.BlockSpec((1, 1, D), lambda e: (e, 0, 0)),
        ],
        out_specs=pl.BlockSpec((T, D), lambda e: (0, 0)),
        out_shape=jax.ShapeDtypeStruct((T, D), jnp.float32),
        scratch_shapes=[
            pltpu.VMEM((T, D), jnp.float32),
            pltpu.VMEM((T, E), jnp.float32),
        ],
        compiler_params=pltpu.CompilerParams(
            dimension_semantics=("arbitrary",)),
    )(xt, Wr, br.reshape(1, E), sW1, sb1.reshape(1, D), sW2, sb2.reshape(1, D),
      rW1, rb1.reshape(E, 1, D), rW2, rb2.reshape(E, 1, D))
    return out.reshape(b, l, d)
